# baseline (speedup 1.0000x reference)
import functools

import jax
import jax.numpy as jnp
from jax import lax
from jax.experimental import pallas as pl
from jax.experimental.pallas import tpu as pltpu
from jax.experimental.pallas import tpu_sc as plsc

B, T = 1024, 20
S_OBS, S_ACT = 8, 2
S_TOT = S_OBS + S_ACT
D = 64
NW = 32
PER = B // NW
N_OBS = T * S_OBS
N_ACT = T * S_ACT
RPB = T * S_TOT // 2
LANES = 16


def _body(obs_tok, act_tok, obs_tab, act_tab, pos_o, pos_a, out, refs):
    (ibo, iba, go, ga, sbuf, pos_ov, pos_av, isem, gsem, osem) = refs
    wid = lax.axis_index("s") * 2 + lax.axis_index("c")
    base_b = wid * PER

    pltpu.sync_copy(pos_o, pos_ov)
    pltpu.sync_copy(pos_a, pos_av)

    def fire_idx(i, p):
        b = base_b + i
        pltpu.async_copy(obs_tok.at[pl.ds(b * N_OBS, N_OBS)], ibo.at[p],
                         isem.at[p])
        pltpu.async_copy(act_tok.at[pl.ds(b * N_ACT, N_ACT)], iba.at[p],
                         isem.at[p])

    def wait_idx(i, p):
        b = base_b + i
        pltpu.make_async_copy(obs_tok.at[pl.ds(b * N_OBS, N_OBS)], ibo.at[p],
                              isem.at[p]).wait()
        pltpu.make_async_copy(act_tok.at[pl.ds(b * N_ACT, N_ACT)], iba.at[p],
                              isem.at[p]).wait()

    def fire_gather(p):
        pltpu.async_copy(obs_tab.at[ibo.at[p, pl.ds(0, 80)]],
                         go.at[p, pl.ds(0, 80), :], gsem.at[p])
        pltpu.async_copy(obs_tab.at[ibo.at[p, pl.ds(80, 80)]],
                         go.at[p, pl.ds(80, 80), :], gsem.at[p])
        pltpu.async_copy(act_tab.at[iba.at[p]], ga.at[p], gsem.at[p])

    def wait_gather(p):
        pltpu.make_async_copy(obs_tab.at[ibo.at[p, pl.ds(0, 80)]],
                              go.at[p, pl.ds(0, 80), :], gsem.at[p]).wait()
        pltpu.make_async_copy(obs_tab.at[ibo.at[p, pl.ds(80, 80)]],
                              go.at[p, pl.ds(80, 80), :], gsem.at[p]).wait()
        pltpu.make_async_copy(act_tab.at[iba.at[p]], ga.at[p],
                              gsem.at[p]).wait()

    def fire_out(i, r):
        b = base_b + i
        pltpu.async_copy(sbuf.at[r], out.at[pl.ds(b * RPB, RPB)], osem.at[r])

    def wait_out(i, r):
        b = base_b + i
        pltpu.make_async_copy(sbuf.at[r], out.at[pl.ds(b * RPB, RPB)],
                              osem.at[r]).wait()

    def add_repack(p, r):
        def add_obs(q, c):
            t = q // S_OBS
            s = q - t * S_OBS
            dr = t * (S_TOT // 2) + s // 2
            dc = (s % 2) * D
            for j in range(D // LANES):
                sl = pl.ds(j * LANES, LANES)
                sbuf[r, dr, pl.ds(dc + j * LANES, LANES)] = (
                    go[p, q, sl] + pos_ov[q, sl])
            return c

        def add_act(q, c):
            t = q // S_ACT
            dr = t * (S_TOT // 2) + S_TOT // 2 - 1
            dc = (q - t * S_ACT) * D
            for j in range(D // LANES):
                sl = pl.ds(j * LANES, LANES)
                sbuf[r, dr, pl.ds(dc + j * LANES, LANES)] = (
                    ga[p, q, sl] + pos_av[q, sl])
            return c

        lax.fori_loop(0, N_OBS, add_obs, 0, unroll=2)
        lax.fori_loop(0, N_ACT, add_act, 0, unroll=2)

    fire_idx(0, 0)
    wait_idx(0, 0)
    fire_gather(0)
    fire_idx(1, 1)
    for i in range(PER):
        p = i % 2
        r = i % 3
        wait_gather(p)
        if i + 1 < PER:
            wait_idx(i + 1, 1 - p)
            fire_gather(1 - p)
        if i + 2 < PER:
            fire_idx(i + 2, p)
        if i >= 2:
            wait_out(i - 2, (i - 2) % 3)
        add_repack(p, r)
        fire_out(i, r)
    wait_out(PER - 2, (PER - 2) % 3)
    wait_out(PER - 1, (PER - 1) % 3)


@functools.partial(
    pl.kernel,
    out_type=jax.ShapeDtypeStruct((B * RPB, 2 * D), jnp.float32),
    mesh=plsc.VectorSubcoreMesh(core_axis_name="c", subcore_axis_name="s",
                                num_cores=2),
    scratch_types=[
        pltpu.VMEM((2, N_OBS), jnp.int32),
        pltpu.VMEM((2, N_ACT), jnp.int32),
        pltpu.VMEM((2, N_OBS, D), jnp.float32),
        pltpu.VMEM((2, N_ACT, D), jnp.float32),
        pltpu.VMEM((3, RPB, 2 * D), jnp.float32),
        pltpu.VMEM((N_OBS, D), jnp.float32),
        pltpu.VMEM((N_ACT, D), jnp.float32),
        pltpu.SemaphoreType.DMA((2,)),
        pltpu.SemaphoreType.DMA((2,)),
        pltpu.SemaphoreType.DMA((3,)),
    ],
    compiler_params=pltpu.CompilerParams(use_tc_tiling_on_sc=False),
)
def _episode_builder(obs_tok, act_tok, obs_tab, act_tab, pos_o, pos_a,
                     out, *refs):
    _body(obs_tok, act_tok, obs_tab, act_tab, pos_o, pos_a, out, refs)


def kernel(obs_tokens, act_tokens, obs_table, act_table, pos_obs, pos_act,
           pos_ts):
    obs_tok = obs_tokens.reshape(B * T * S_OBS).astype(jnp.int32)
    act_tok = act_tokens.reshape(B * T * S_ACT).astype(jnp.int32)
    pos_o = (pos_obs[None, :, :] + pos_ts[:, None, :]).reshape(N_OBS, D)
    pos_a = (pos_act[None, :, :] + pos_ts[:, None, :]).reshape(N_ACT, D)
    out = _episode_builder(obs_tok, act_tok, obs_table, act_table,
                           pos_o, pos_a)
    return out.reshape(B, T * S_TOT, D)


# speedup vs baseline: 1.0081x; 1.0081x over previous
import functools

import jax
import jax.numpy as jnp
from jax import lax
from jax.experimental import pallas as pl
from jax.experimental.pallas import tpu as pltpu
from jax.experimental.pallas import tpu_sc as plsc

B, T = 1024, 20
S_OBS, S_ACT = 8, 2
S_TOT = S_OBS + S_ACT
D = 64
NW = 32
PER = B // NW
N_OBS = T * S_OBS
N_ACT = T * S_ACT
NBT = 8


def _sc_body(obs_tok, act_tok, obs_tab, act_tab, og, ag, refs):
    (ibo, iba, go, ga, isem, gsem, osem) = refs
    wid = lax.axis_index("s") * 2 + lax.axis_index("c")
    base_b = wid * PER

    def fire_idx(i, p):
        b = base_b + i
        pltpu.async_copy(obs_tok.at[pl.ds(b * N_OBS, N_OBS)], ibo.at[p],
                         isem.at[p])
        pltpu.async_copy(act_tok.at[pl.ds(b * N_ACT, N_ACT)], iba.at[p],
                         isem.at[p])

    def wait_idx(i, p):
        b = base_b + i
        pltpu.make_async_copy(obs_tok.at[pl.ds(b * N_OBS, N_OBS)], ibo.at[p],
                              isem.at[p]).wait()
        pltpu.make_async_copy(act_tok.at[pl.ds(b * N_ACT, N_ACT)], iba.at[p],
                              isem.at[p]).wait()

    def fire_gather(p):
        pltpu.async_copy(obs_tab.at[ibo.at[p, pl.ds(0, 80)]],
                         go.at[p, pl.ds(0, 80), :], gsem.at[p])
        pltpu.async_copy(obs_tab.at[ibo.at[p, pl.ds(80, 80)]],
                         go.at[p, pl.ds(80, 80), :], gsem.at[p])
        pltpu.async_copy(act_tab.at[iba.at[p]], ga.at[p], gsem.at[p])

    def wait_gather(p):
        pltpu.make_async_copy(obs_tab.at[ibo.at[p, pl.ds(0, 80)]],
                              go.at[p, pl.ds(0, 80), :], gsem.at[p]).wait()
        pltpu.make_async_copy(obs_tab.at[ibo.at[p, pl.ds(80, 80)]],
                              go.at[p, pl.ds(80, 80), :], gsem.at[p]).wait()
        pltpu.make_async_copy(act_tab.at[iba.at[p]], ga.at[p],
                              gsem.at[p]).wait()

    def fire_out(i, p):
        b = base_b + i
        pltpu.async_copy(go.at[p], og.at[pl.ds(b * N_OBS, N_OBS)], osem.at[p])
        pltpu.async_copy(ga.at[p], ag.at[pl.ds(b * N_ACT, N_ACT)], osem.at[p])

    def wait_out(i, p):
        b = base_b + i
        pltpu.make_async_copy(go.at[p], og.at[pl.ds(b * N_OBS, N_OBS)],
                              osem.at[p]).wait()
        pltpu.make_async_copy(ga.at[p], ag.at[pl.ds(b * N_ACT, N_ACT)],
                              osem.at[p]).wait()

    fire_idx(0, 0)
    wait_idx(0, 0)
    fire_gather(0)
    fire_idx(1, 1)
    for i in range(PER):
        p = i % 2
        wait_gather(p)
        if i + 1 < PER:
            wait_idx(i + 1, 1 - p)
            if i >= 1:
                wait_out(i - 1, 1 - p)
            fire_gather(1 - p)
        if i + 2 < PER:
            fire_idx(i + 2, p)
        fire_out(i, p)
    wait_out(PER - 1, (PER - 1) % 2)


@functools.partial(
    pl.kernel,
    out_type=(jax.ShapeDtypeStruct((B * N_OBS, D), jnp.float32),
              jax.ShapeDtypeStruct((B * N_ACT, D), jnp.float32)),
    mesh=plsc.VectorSubcoreMesh(core_axis_name="c", subcore_axis_name="s",
                                num_cores=2),
    scratch_types=[
        pltpu.VMEM((2, N_OBS), jnp.int32),
        pltpu.VMEM((2, N_ACT), jnp.int32),
        pltpu.VMEM((2, N_OBS, D), jnp.float32),
        pltpu.VMEM((2, N_ACT, D), jnp.float32),
        pltpu.SemaphoreType.DMA((2,)),
        pltpu.SemaphoreType.DMA((2,)),
        pltpu.SemaphoreType.DMA((2,)),
    ],
    compiler_params=pltpu.CompilerParams(use_tc_tiling_on_sc=False),
)
def _sc_gather(obs_tok, act_tok, obs_tab, act_tab, og, ag, *refs):
    _sc_body(obs_tok, act_tok, obs_tab, act_tab, og, ag, refs)


def _tc_body(og_ref, ag_ref, po_ref, pa_ref, out_ref):
    o = og_ref[...].reshape(NBT, N_OBS // 2, 2 * D) + po_ref[...][None]
    a = ag_ref[...].reshape(NBT, N_ACT // 2, 2 * D) + pa_ref[...][None]
    o4 = o.reshape(NBT, T, S_OBS // 2, 2 * D)
    a4 = a.reshape(NBT, T, S_ACT // 2, 2 * D)
    x = jnp.concatenate([o4, a4], axis=2)
    lo = x[..., 0:D]
    hi = x[..., D:2 * D]
    y = jnp.concatenate([lo[..., None, :], hi[..., None, :]], axis=-2)
    out_ref[...] = y.reshape(NBT, T * S_TOT, D)


@functools.partial(
    pl.pallas_call,
    out_shape=jax.ShapeDtypeStruct((B, T * S_TOT, D), jnp.float32),
    grid=(B // NBT,),
    in_specs=[
        pl.BlockSpec((NBT * N_OBS // 2, 2 * D), lambda i: (i, 0)),
        pl.BlockSpec((NBT * N_ACT // 2, 2 * D), lambda i: (i, 0)),
        pl.BlockSpec((N_OBS // 2, 2 * D), lambda i: (0, 0)),
        pl.BlockSpec((N_ACT // 2, 2 * D), lambda i: (0, 0)),
    ],
    out_specs=pl.BlockSpec((NBT, T * S_TOT, D), lambda i: (i, 0, 0)),
    compiler_params=pltpu.CompilerParams(
        dimension_semantics=("parallel",)),
)
def _tc_pack(og_ref, ag_ref, po_ref, pa_ref, out_ref):
    _tc_body(og_ref, ag_ref, po_ref, pa_ref, out_ref)


def kernel(obs_tokens, act_tokens, obs_table, act_table, pos_obs, pos_act,
           pos_ts):
    obs_tok = obs_tokens.reshape(B * T * S_OBS).astype(jnp.int32)
    act_tok = act_tokens.reshape(B * T * S_ACT).astype(jnp.int32)
    og, ag = _sc_gather(obs_tok, act_tok, obs_table, act_table)
    og2 = og.reshape(B * N_OBS // 2, 2 * D)
    ag2 = ag.reshape(B * N_ACT // 2, 2 * D)
    pos_o = (pos_obs[None, :, :] + pos_ts[:, None, :]).reshape(
        N_OBS // 2, 2 * D)
    pos_a = (pos_act[None, :, :] + pos_ts[:, None, :]).reshape(
        N_ACT // 2, 2 * D)
    return _tc_pack(og2, ag2, pos_o, pos_a)
